# Initial kernel scaffold; baseline (speedup 1.0000x reference)
import jax
import jax.numpy as jnp
from jax.experimental import pallas as pl


def kernel(hidden_states, router_W, gate_W, gate_b, up_W, up_b, down_W, down_b):
    raise NotImplementedError("write your pallas kernel here")


# speedup vs baseline: 2.9616x; 2.9616x over previous
import functools

import jax
import jax.numpy as jnp
from jax.experimental import pallas as pl
from jax.experimental.pallas import tpu as pltpu

AUXW = 0.01


def _router_body(x_ref, rw_ref, wf_ref, aux_ref, *, n, e):
    x = x_ref[...]
    rw = rw_ref[...]
    logits = jax.lax.dot_general(x, rw, (((1,), (1,)), ((), ())),
                                 preferred_element_type=jnp.float32)
    m = jnp.max(logits, axis=1, keepdims=True)
    ex = jnp.exp(logits - m)
    p = ex / jnp.sum(ex, axis=1, keepdims=True)
    iota = jax.lax.broadcasted_iota(jnp.int32, (n, e), 1)
    m1 = jnp.max(p, axis=1, keepdims=True)
    i1 = jnp.min(jnp.where(p == m1, iota, e), axis=1, keepdims=True)
    sel1 = iota == i1
    pm = jnp.where(sel1, -1.0, p)
    m2 = jnp.max(pm, axis=1, keepdims=True)
    i2 = jnp.min(jnp.where(pm == m2, iota, e), axis=1, keepdims=True)
    sel2 = iota == i2
    s = m1 + m2
    wf = jnp.where(sel1, m1 / s, 0.0) + jnp.where(sel2, m2 / s, 0.0)
    wf_ref[...] = wf
    col = jnp.sum(wf, axis=0, keepdims=True)
    aux_ref[...] = jnp.sum(col * col, keepdims=True).reshape(1, 1) * (AUXW / n)


def _ffn_body(x_ref, wf_ref, gw_ref, gb_ref, uw_ref, ub_ref, dw_ref, db_ref,
              out_ref, *, n, e, nib):
    ei = pl.program_id(0)
    ii = pl.program_id(1)

    @pl.when((ei == 0) & (ii == 0))
    def _():
        out_ref[...] = jnp.zeros_like(out_ref)

    iota = jax.lax.broadcasted_iota(jnp.int32, wf_ref.shape, 1)
    wcol = jnp.sum(jnp.where(iota == ei, wf_ref[...], 0.0), axis=1,
                   keepdims=True)
    active = jnp.max(wcol) > 0.0

    @pl.when(active)
    def _():
        x = x_ref[...]
        g = jax.lax.dot_general(x, gw_ref[0], (((1,), (1,)), ((), ())),
                                preferred_element_type=jnp.float32)
        g = g + gb_ref[0, 0]
        u = jax.lax.dot_general(x, uw_ref[0], (((1,), (1,)), ((), ())),
                                preferred_element_type=jnp.float32)
        u = u + ub_ref[0, 0]
        a = g * jax.nn.sigmoid(g) * u
        part = jax.lax.dot_general(a, dw_ref[0], (((1,), (1,)), ((), ())),
                                   preferred_element_type=jnp.float32)
        acc = wcol * part

        @pl.when(ii == 0)
        def _():
            out_ref[...] = out_ref[...] + acc + wcol * db_ref[0]

        @pl.when(ii != 0)
        def _():
            out_ref[...] = out_ref[...] + acc


def kernel(hidden_states, router_W, gate_W, gate_b, up_W, up_b, down_W,
           down_b):
    b, s, h = hidden_states.shape
    e, i_dim = gate_W.shape[:2]
    n = b * s
    x = hidden_states.reshape(n, h)

    wf, aux = pl.pallas_call(
        functools.partial(_router_body, n=n, e=e),
        out_shape=(
            jax.ShapeDtypeStruct((n, e), jnp.float32),
            jax.ShapeDtypeStruct((1, 1), jnp.float32),
        ),
    )(x, router_W)

    ib = 512
    nib = i_dim // ib
    grid = (e, nib)
    combined = pl.pallas_call(
        functools.partial(_ffn_body, n=n, e=e, nib=nib),
        grid=grid,
        in_specs=[
            pl.BlockSpec((n, h), lambda ei, ii: (0, 0)),
            pl.BlockSpec((n, e), lambda ei, ii: (0, 0)),
            pl.BlockSpec((1, ib, h), lambda ei, ii: (ei, ii, 0)),
            pl.BlockSpec((1, 1, 1, ib), lambda ei, ii: (ei, ii, 0, 0)),
            pl.BlockSpec((1, ib, h), lambda ei, ii: (ei, ii, 0)),
            pl.BlockSpec((1, 1, 1, ib), lambda ei, ii: (ei, ii, 0, 0)),
            pl.BlockSpec((1, h, ib), lambda ei, ii: (ei, 0, ii)),
            pl.BlockSpec((1, 1, h), lambda ei, ii: (ei, 0, 0)),
        ],
        out_specs=pl.BlockSpec((n, h), lambda ei, ii: (0, 0)),
        out_shape=jax.ShapeDtypeStruct((n, h), jnp.float32),
        compiler_params=pltpu.CompilerParams(
            dimension_semantics=("arbitrary", "arbitrary")),
    )(x, wf, gate_W, gate_b.reshape(e, nib, 1, ib), up_W,
      up_b.reshape(e, nib, 1, ib), down_W, down_b.reshape(e, 1, h))

    return combined.reshape(b, s, h), aux[0, 0]


# speedup vs baseline: 4.3151x; 1.4570x over previous
import functools

import jax
import jax.numpy as jnp
from jax.experimental import pallas as pl
from jax.experimental.pallas import tpu as pltpu

AUXW = 0.01


def _router_body(x_ref, rw_ref, wf_ref, scol_ref, aux_ref, *, n, e):
    x = x_ref[...]
    rw = rw_ref[...]
    logits = jax.lax.dot_general(x, rw, (((1,), (1,)), ((), ())),
                                 preferred_element_type=jnp.float32)
    m = jnp.max(logits, axis=1, keepdims=True)
    ex = jnp.exp(logits - m)
    p = ex / jnp.sum(ex, axis=1, keepdims=True)
    iota = jax.lax.broadcasted_iota(jnp.int32, (n, e), 1)
    m1 = jnp.max(p, axis=1, keepdims=True)
    i1 = jnp.min(jnp.where(p == m1, iota, e), axis=1, keepdims=True)
    sel1 = iota == i1
    pm = jnp.where(sel1, -1.0, p)
    m2 = jnp.max(pm, axis=1, keepdims=True)
    i2 = jnp.min(jnp.where(pm == m2, iota, e), axis=1, keepdims=True)
    sel2 = iota == i2
    s = m1 + m2
    wf = jnp.where(sel1, m1 / s, 0.0) + jnp.where(sel2, m2 / s, 0.0)
    wf_ref[...] = wf
    col = jnp.sum(wf, axis=0, keepdims=True)
    scol_ref[...] = col
    aux_ref[...] = jnp.sum(col * col, keepdims=True).reshape(1, 1) * (AUXW / n)


def _ffn_body(eo_ref, na_ref, x_ref, wf_ref, gw_ref, gb_ref, uw_ref, ub_ref,
              dw_ref, db_ref, out_ref, *, n, e, nib):
    ei = pl.program_id(0)
    ii = pl.program_id(1)

    @pl.when((ei == 0) & (ii == 0))
    def _():
        out_ref[...] = jnp.zeros_like(out_ref)

    @pl.when(ei < na_ref[0])
    def _():
        emap = eo_ref[jnp.minimum(ei, na_ref[0] - 1)]
        iota = jax.lax.broadcasted_iota(jnp.int32, wf_ref.shape, 1)
        wcol = jnp.sum(jnp.where(iota == emap, wf_ref[...], 0.0), axis=1,
                       keepdims=True)
        x = x_ref[...]
        gw = gw_ref[0].astype(jnp.bfloat16)
        uw = uw_ref[0].astype(jnp.bfloat16)
        dw = dw_ref[0].astype(jnp.bfloat16)
        g = jax.lax.dot_general(x, gw, (((1,), (1,)), ((), ())),
                                preferred_element_type=jnp.float32)
        g = g + gb_ref[0, 0]
        u = jax.lax.dot_general(x, uw, (((1,), (1,)), ((), ())),
                                preferred_element_type=jnp.float32)
        u = u + ub_ref[0, 0]
        a = (g * jax.nn.sigmoid(g) * u).astype(jnp.bfloat16)
        part = jax.lax.dot_general(a, dw, (((1,), (1,)), ((), ())),
                                   preferred_element_type=jnp.float32)
        acc = wcol * part

        @pl.when(ii == 0)
        def _():
            out_ref[...] = out_ref[...] + acc + wcol * db_ref[0]

        @pl.when(ii != 0)
        def _():
            out_ref[...] = out_ref[...] + acc


def kernel(hidden_states, router_W, gate_W, gate_b, up_W, up_b, down_W,
           down_b):
    b, s, h = hidden_states.shape
    e, i_dim = gate_W.shape[:2]
    n = b * s
    x = hidden_states.reshape(n, h)

    wf, scol, aux = pl.pallas_call(
        functools.partial(_router_body, n=n, e=e),
        out_shape=(
            jax.ShapeDtypeStruct((n, e), jnp.float32),
            jax.ShapeDtypeStruct((1, e), jnp.float32),
            jax.ShapeDtypeStruct((1, 1), jnp.float32),
        ),
    )(x, router_W)

    iota8 = jnp.arange(e, dtype=jnp.int32)
    active = scol[0] > 0.0
    eorder = jnp.argsort(jnp.where(active, iota8, iota8 + e)).astype(jnp.int32)
    nact = jnp.sum(active.astype(jnp.int32)).reshape(1)

    ib = 512
    nib = i_dim // ib
    xb = x.astype(jnp.bfloat16)

    def wspec_in(ei, ii, eo, na):
        act = ei < na[0]
        eix = eo[jnp.where(act, ei, na[0] - 1)]
        iix = jnp.where(act, ii, nib - 1)
        return eix, iix

    grid_spec = pltpu.PrefetchScalarGridSpec(
        num_scalar_prefetch=2,
        grid=(e, nib),
        in_specs=[
            pl.BlockSpec((n, h), lambda ei, ii, eo, na: (0, 0)),
            pl.BlockSpec((n, e), lambda ei, ii, eo, na: (0, 0)),
            pl.BlockSpec((1, ib, h),
                         lambda ei, ii, eo, na: (*wspec_in(ei, ii, eo, na), 0)),
            pl.BlockSpec((1, 1, 1, ib),
                         lambda ei, ii, eo, na: (*wspec_in(ei, ii, eo, na), 0, 0)),
            pl.BlockSpec((1, ib, h),
                         lambda ei, ii, eo, na: (*wspec_in(ei, ii, eo, na), 0)),
            pl.BlockSpec((1, 1, 1, ib),
                         lambda ei, ii, eo, na: (*wspec_in(ei, ii, eo, na), 0, 0)),
            pl.BlockSpec((1, h, ib),
                         lambda ei, ii, eo, na:
                         (wspec_in(ei, ii, eo, na)[0], 0,
                          wspec_in(ei, ii, eo, na)[1])),
            pl.BlockSpec((1, 1, h),
                         lambda ei, ii, eo, na:
                         (wspec_in(ei, ii, eo, na)[0], 0, 0)),
        ],
        out_specs=pl.BlockSpec((n, h), lambda ei, ii, eo, na: (0, 0)),
    )

    combined = pl.pallas_call(
        functools.partial(_ffn_body, n=n, e=e, nib=nib),
        grid_spec=grid_spec,
        out_shape=jax.ShapeDtypeStruct((n, h), jnp.float32),
        compiler_params=pltpu.CompilerParams(
            dimension_semantics=("arbitrary", "arbitrary")),
    )(eorder, nact, xb, wf, gate_W, gate_b.reshape(e, nib, 1, ib), up_W,
      up_b.reshape(e, nib, 1, ib), down_W, down_b.reshape(e, 1, h))

    return combined.reshape(b, s, h), aux[0, 0]


# speedup vs baseline: 4.6005x; 1.0661x over previous
import functools

import jax
import jax.numpy as jnp
from jax.experimental import pallas as pl
from jax.experimental.pallas import tpu as pltpu

AUXW = 0.01


def _router_body(x_ref, rw_ref, wf_ref, scol_ref, aux_ref, *, n, e):
    x = x_ref[...]
    rw = rw_ref[...]
    logits = jax.lax.dot_general(x, rw, (((1,), (1,)), ((), ())),
                                 preferred_element_type=jnp.float32)
    m = jnp.max(logits, axis=1, keepdims=True)
    ex = jnp.exp(logits - m)
    p = ex / jnp.sum(ex, axis=1, keepdims=True)
    iota = jax.lax.broadcasted_iota(jnp.int32, (n, e), 1)
    m1 = jnp.max(p, axis=1, keepdims=True)
    i1 = jnp.min(jnp.where(p == m1, iota, e), axis=1, keepdims=True)
    sel1 = iota == i1
    pm = jnp.where(sel1, -1.0, p)
    m2 = jnp.max(pm, axis=1, keepdims=True)
    i2 = jnp.min(jnp.where(pm == m2, iota, e), axis=1, keepdims=True)
    sel2 = iota == i2
    s = m1 + m2
    wf = jnp.where(sel1, m1 / s, 0.0) + jnp.where(sel2, m2 / s, 0.0)
    wf_ref[...] = wf
    col = jnp.sum(wf, axis=0, keepdims=True)
    scol_ref[...] = col
    aux_ref[...] = jnp.sum(col * col, keepdims=True).reshape(1, 1) * (AUXW / n)


def _ffn_body(eo_ref, na_ref, x_ref, wf_ref, gw_ref, gb_ref, uw_ref, ub_ref,
              dw_ref, db_ref, out_ref, wcol_ref, *, n, e, nib):
    ei = pl.program_id(0)
    ii = pl.program_id(1)

    @pl.when((ei == 0) & (ii == 0))
    def _():
        out_ref[...] = jnp.zeros_like(out_ref)

    @pl.when(ei < na_ref[0])
    def _():
        emap = eo_ref[jnp.minimum(ei, na_ref[0] - 1)]

        @pl.when(ii == 0)
        def _():
            onehot = (jax.lax.broadcasted_iota(jnp.int32, (e, 1), 0)
                      == emap).astype(jnp.float32)
            wcol_ref[...] = jax.lax.dot_general(
                wf_ref[...], onehot, (((1,), (0,)), ((), ())),
                preferred_element_type=jnp.float32)
            out_ref[...] = out_ref[...] + wcol_ref[...] * db_ref[0]

        wcol = wcol_ref[...]
        x = x_ref[...]
        gw = gw_ref[0].astype(jnp.bfloat16)
        uw = uw_ref[0].astype(jnp.bfloat16)
        dw = dw_ref[0].astype(jnp.bfloat16)
        g = jax.lax.dot_general(x, gw, (((1,), (1,)), ((), ())),
                                preferred_element_type=jnp.float32)
        g = g + gb_ref[0, 0]
        u = jax.lax.dot_general(x, uw, (((1,), (1,)), ((), ())),
                                preferred_element_type=jnp.float32)
        u = u + ub_ref[0, 0]
        a = (g * jax.nn.sigmoid(g) * (u * wcol)).astype(jnp.bfloat16)
        part = jax.lax.dot_general(a, dw, (((1,), (1,)), ((), ())),
                                   preferred_element_type=jnp.float32)
        out_ref[...] = out_ref[...] + part


def kernel(hidden_states, router_W, gate_W, gate_b, up_W, up_b, down_W,
           down_b):
    b, s, h = hidden_states.shape
    e, i_dim = gate_W.shape[:2]
    n = b * s
    x = hidden_states.reshape(n, h)

    wf, scol, aux = pl.pallas_call(
        functools.partial(_router_body, n=n, e=e),
        out_shape=(
            jax.ShapeDtypeStruct((n, e), jnp.float32),
            jax.ShapeDtypeStruct((1, e), jnp.float32),
            jax.ShapeDtypeStruct((1, 1), jnp.float32),
        ),
    )(x, router_W)

    iota8 = jnp.arange(e, dtype=jnp.int32)
    active = scol[0] > 0.0
    eorder = jnp.argsort(jnp.where(active, iota8, iota8 + e)).astype(jnp.int32)
    nact = jnp.sum(active.astype(jnp.int32)).reshape(1)

    ib = 512
    nib = i_dim // ib
    xb = x.astype(jnp.bfloat16)

    def wspec_in(ei, ii, eo, na):
        act = ei < na[0]
        eix = eo[jnp.where(act, ei, na[0] - 1)]
        iix = jnp.where(act, ii, nib - 1)
        return eix, iix

    grid_spec = pltpu.PrefetchScalarGridSpec(
        num_scalar_prefetch=2,
        grid=(e, nib),
        in_specs=[
            pl.BlockSpec((n, h), lambda ei, ii, eo, na: (0, 0)),
            pl.BlockSpec((n, e), lambda ei, ii, eo, na: (0, 0)),
            pl.BlockSpec((1, ib, h),
                         lambda ei, ii, eo, na: (*wspec_in(ei, ii, eo, na), 0)),
            pl.BlockSpec((1, 1, 1, ib),
                         lambda ei, ii, eo, na: (*wspec_in(ei, ii, eo, na), 0, 0)),
            pl.BlockSpec((1, ib, h),
                         lambda ei, ii, eo, na: (*wspec_in(ei, ii, eo, na), 0)),
            pl.BlockSpec((1, 1, 1, ib),
                         lambda ei, ii, eo, na: (*wspec_in(ei, ii, eo, na), 0, 0)),
            pl.BlockSpec((1, h, ib),
                         lambda ei, ii, eo, na:
                         (wspec_in(ei, ii, eo, na)[0], 0,
                          wspec_in(ei, ii, eo, na)[1])),
            pl.BlockSpec((1, 1, h),
                         lambda ei, ii, eo, na:
                         (wspec_in(ei, ii, eo, na)[0], 0, 0)),
        ],
        out_specs=pl.BlockSpec((n, h), lambda ei, ii, eo, na: (0, 0)),
        scratch_shapes=[pltpu.VMEM((n, 1), jnp.float32)],
    )

    combined = pl.pallas_call(
        functools.partial(_ffn_body, n=n, e=e, nib=nib),
        grid_spec=grid_spec,
        out_shape=jax.ShapeDtypeStruct((n, h), jnp.float32),
        compiler_params=pltpu.CompilerParams(
            dimension_semantics=("arbitrary", "arbitrary")),
    )(eorder, nact, xb, wf, gate_W, gate_b.reshape(e, nib, 1, ib), up_W,
      up_b.reshape(e, nib, 1, ib), down_W, down_b.reshape(e, 1, h))

    return combined.reshape(b, s, h), aux[0, 0]
